# baseline (speedup 1.0000x reference)
import functools
import math

import jax
import jax.numpy as jnp
from jax.experimental import pallas as pl
from jax.experimental.pallas import tpu as pltpu
from jax.experimental.pallas import tpu_sc as plsc

_R_BLK = 384
_NC, _NS = 2, 16
_ROWS_PER_W = 8


def _sc_gather_body(table_hbm, idx_hbm, out_hbm, idx_v, rows_v, sem,
                    *, n_active):
    wid = jax.lax.axis_index("s") * _NC + jax.lax.axis_index("c")

    @pl.when(wid < n_active)
    def _():
        base = wid * _ROWS_PER_W
        pltpu.sync_copy(idx_hbm.at[pl.ds(base, _ROWS_PER_W)], idx_v)
        pltpu.async_copy(table_hbm.at[idx_v], rows_v, sem).wait()
        pltpu.sync_copy(rows_v, out_hbm.at[pl.ds(base, _ROWS_PER_W)])


def _month_gather_sc(month_table, months_flat):
    table = jnp.pad(month_table, ((0, 0), (0, 128 - month_table.shape[1])))
    n = months_flat.shape[0]
    n_active = n // _ROWS_PER_W
    mesh = plsc.VectorSubcoreMesh(core_axis_name="c", subcore_axis_name="s")
    return pl.kernel(
        functools.partial(_sc_gather_body, n_active=n_active),
        out_type=jax.ShapeDtypeStruct((n, 128), jnp.float32),
        mesh=mesh,
        scratch_types=[
            pltpu.VMEM((_ROWS_PER_W,), jnp.int32),
            pltpu.VMEM((_ROWS_PER_W, 128), jnp.float32),
            pltpu.SemaphoreType.DMA,
        ],
    )(table, months_flat)


def _tc_body(me_ref, ce_ref, pe_ref, ratio_ref, tok_ref, out_ref,
             *, t, s, w, d4, r_blk):
    half = d4 // 4

    pe = pe_ref[:t, :]
    me = me_ref[0][:, :d4]
    zt = jnp.zeros((t, d4), dtype=jnp.float32)
    a_t = jnp.concatenate([zt, pe, me, zt], axis=-1)

    r0 = (pl.program_id(1) * r_blk
          + jax.lax.broadcasted_iota(jnp.int32, (r_blk, 1), 0))
    s_idx = jax.lax.rem(r0, s)
    oh_s = (s_idx == jax.lax.broadcasted_iota(jnp.int32, (r_blk, s), 1)
            ).astype(jnp.float32)
    ch = jnp.dot(oh_s, ce_ref[:, :], preferred_element_type=jnp.float32)

    ratio = ratio_ref[0, 0]
    hw = jax.lax.div(r0, s)
    hpos = jax.lax.div(hw, w).astype(jnp.float32) * ratio
    wpos = jax.lax.rem(hw, w).astype(jnp.float32) * ratio
    k = jax.lax.broadcasted_iota(jnp.int32, (1, half), 1).astype(jnp.float32)
    omega = jnp.exp(k * (-math.log(10000.0) / half))
    ah = hpos * omega
    aw = wpos * omega
    a_r = jnp.concatenate([
        ch, jnp.zeros((r_blk, 2 * d4), dtype=jnp.float32),
        jnp.sin(ah), jnp.cos(ah), jnp.sin(aw), jnp.cos(aw),
    ], axis=-1)

    out_ref[...] = (tok_ref[...]
                    + a_t[None, None, :, :]
                    + a_r[None, :, None, :])


def kernel(tokens, timestamps, patch_size, input_res, channel_embed,
           pos_embed, month_table):
    b, h, w, t, s, d = tokens.shape
    d4 = d // 4
    r_blk = _R_BLK
    rows = h * w * s
    tok4 = tokens.transpose(0, 1, 2, 4, 3, 5).reshape(b, rows, t, d)
    months_flat = timestamps[:, :, 1].astype(jnp.int32).reshape(b * t)
    me = _month_gather_sc(month_table, months_flat).reshape(b, t, 128)
    ratio = (jnp.float32(input_res) * jnp.float32(patch_size) / 10.0
             ).reshape(1, 1)

    grid = (b, rows // r_blk)
    out = pl.pallas_call(
        functools.partial(_tc_body, t=t, s=s, w=w, d4=d4, r_blk=r_blk),
        grid=grid,
        in_specs=[
            pl.BlockSpec((1, t, 128), lambda i, j: (i, 0, 0)),
            pl.BlockSpec(channel_embed.shape, lambda i, j: (0, 0)),
            pl.BlockSpec(pos_embed.shape, lambda i, j: (0, 0)),
            pl.BlockSpec(memory_space=pltpu.SMEM),
            pl.BlockSpec((1, r_blk, t, d), lambda i, j: (i, j, 0, 0)),
        ],
        out_specs=pl.BlockSpec((1, r_blk, t, d), lambda i, j: (i, j, 0, 0)),
        out_shape=jax.ShapeDtypeStruct((b, rows, t, d), jnp.float32),
        compiler_params=pltpu.CompilerParams(
            dimension_semantics=("parallel", "parallel")),
    )(me, channel_embed, pos_embed, ratio, tok4)
    return out.reshape(b, h, w, s, t, d).transpose(0, 1, 2, 4, 3, 5)


# speedup vs baseline: 1.1772x; 1.1772x over previous
import functools
import math

import jax
import jax.numpy as jnp
from jax.experimental import pallas as pl
from jax.experimental.pallas import tpu as pltpu

_R_BLK = 384


def _body(months_ref, ce_ref, pe_ref, mt_ref, ratio_ref, tok_ref, out_ref,
          *, t, s, w, d4, r_blk):
    half = d4 // 4

    pe = pe_ref[:t, :]
    m = months_ref[0]
    oh = (m == jax.lax.broadcasted_iota(jnp.int32, (t, 12), 1)).astype(jnp.float32)
    me = jnp.dot(oh, mt_ref[:, :], preferred_element_type=jnp.float32)
    zt = jnp.zeros((t, d4), dtype=jnp.float32)
    a_t = jnp.concatenate([zt, pe, me, zt], axis=-1)

    r0 = (pl.program_id(1) * r_blk
          + jax.lax.broadcasted_iota(jnp.int32, (r_blk, 1), 0))
    s_idx = jax.lax.rem(r0, s)
    oh_s = (s_idx == jax.lax.broadcasted_iota(jnp.int32, (r_blk, s), 1)
            ).astype(jnp.float32)
    ch = jnp.dot(oh_s, ce_ref[:, :], preferred_element_type=jnp.float32)

    ratio = ratio_ref[0, 0]
    hw = jax.lax.div(r0, s)
    hpos = jax.lax.div(hw, w).astype(jnp.float32) * ratio
    wpos = jax.lax.rem(hw, w).astype(jnp.float32) * ratio
    k = jax.lax.broadcasted_iota(jnp.int32, (1, half), 1).astype(jnp.float32)
    omega = jnp.exp(k * (-math.log(10000.0) / half))
    ah = hpos * omega
    aw = wpos * omega
    a_r = jnp.concatenate([
        ch, jnp.zeros((r_blk, 2 * d4), dtype=jnp.float32),
        jnp.sin(ah), jnp.cos(ah), jnp.sin(aw), jnp.cos(aw),
    ], axis=-1)

    out_ref[...] = (tok_ref[...]
                    + a_t[None, None, :, :]
                    + a_r[None, :, None, :])


def kernel(tokens, timestamps, patch_size, input_res, channel_embed,
           pos_embed, month_table):
    b, h, w, t, s, d = tokens.shape
    d4 = d // 4
    r_blk = _R_BLK
    rows = h * w * s
    tok4 = tokens.transpose(0, 1, 2, 4, 3, 5).reshape(b, rows, t, d)
    months = timestamps[:, :, 1].astype(jnp.int32).reshape(b, t, 1)
    ratio = (jnp.float32(input_res) * jnp.float32(patch_size) / 10.0
             ).reshape(1, 1)

    grid = (b, rows // r_blk)
    out = pl.pallas_call(
        functools.partial(_body, t=t, s=s, w=w, d4=d4, r_blk=r_blk),
        grid=grid,
        in_specs=[
            pl.BlockSpec((1, t, 1), lambda i, j: (i, 0, 0)),
            pl.BlockSpec(channel_embed.shape, lambda i, j: (0, 0)),
            pl.BlockSpec(pos_embed.shape, lambda i, j: (0, 0)),
            pl.BlockSpec(month_table.shape, lambda i, j: (0, 0)),
            pl.BlockSpec(memory_space=pltpu.SMEM),
            pl.BlockSpec((1, r_blk, t, d), lambda i, j: (i, j, 0, 0)),
        ],
        out_specs=pl.BlockSpec((1, r_blk, t, d), lambda i, j: (i, j, 0, 0)),
        out_shape=jax.ShapeDtypeStruct((b, rows, t, d), jnp.float32),
        compiler_params=pltpu.CompilerParams(
            dimension_semantics=("parallel", "parallel")),
    )(months, channel_embed, pos_embed, month_table, ratio, tok4)
    return out.reshape(b, h, w, s, t, d).transpose(0, 1, 2, 4, 3, 5)
